# baseline (speedup 1.0000x reference)
import functools

import jax
import jax.numpy as jnp
from jax import lax
from jax.experimental import pallas as pl
from jax.experimental.pallas import tpu as pltpu
from jax.experimental.pallas import tpu_sc as plsc

VOCAB = 100000
HIDDEN = 1024
BATCH = 4
SEQ = 4096

_B = BATCH * SEQ
_CHUNK = 32
_NB = 3


def _make_gather(V, D, B, CHUNK, NB):
    info = plsc.get_sparse_core_info()
    NC, NS = info.num_cores, info.num_subcores
    NW = NC * NS
    assert B % NW == 0
    b_per_w = B // NW
    n_steps = b_per_w // CHUNK
    n_blocks = n_steps // NB
    assert b_per_w % CHUNK == 0 and NB >= 2
    w_per_row = SEQ // b_per_w
    assert SEQ % b_per_w == 0
    mesh = plsc.VectorSubcoreMesh(core_axis_name="c", subcore_axis_name="s")

    @functools.partial(
        pl.kernel,
        mesh=mesh,
        out_type=jax.ShapeDtypeStruct((BATCH, SEQ, D), jnp.float32),
        scratch_types=[
            pltpu.VMEM((b_per_w,), jnp.int32),
            [pltpu.VMEM((CHUNK, D), jnp.float32) for _ in range(NB)],
            [pltpu.SemaphoreType.DMA for _ in range(NB)],
            [pltpu.SemaphoreType.DMA for _ in range(NB)],
        ],
    )
    def gather_kernel(ids_hbm, table_hbm, out_hbm, idx_v, bufs, gsems, ssems):
        wid = lax.axis_index("s") * NC + lax.axis_index("c")
        row = wid // w_per_row
        col = (wid % w_per_row) * b_per_w
        pltpu.sync_copy(ids_hbm.at[row, pl.ds(col, b_per_w)], idx_v)

        def gather_to(g, p):
            pltpu.async_copy(
                table_hbm.at[idx_v.at[pl.ds(g * CHUNK, CHUNK)]], bufs[p], gsems[p]
            )

        def store_descs(g, p):
            h = CHUNK // 2
            return [
                pltpu.make_async_copy(
                    bufs[p].at[pl.ds(k * h, h)],
                    out_hbm.at[row, pl.ds(col + g * CHUNK + k * h, h)],
                    ssems[p],
                )
                for k in range(2)
            ]

        def store_desc(g, p):
            class _Pair:
                def start(self):
                    for d in store_descs(g, p):
                        d.start()

                def wait(self):
                    for d in store_descs(g, p):
                        d.wait()

            return _Pair()

        for p in range(NB - 1):
            gather_to(p, p)

        def block(i, carry):
            for p in range(NB):
                g = i * NB + p
                q = (p - 1) % NB

                @pl.when(g >= 1)
                def _():
                    store_desc(g - 1, q).wait()

                @pl.when(g + NB - 1 < n_steps)
                def _():
                    gather_to(g + NB - 1, q)

                pltpu.make_async_copy(
                    table_hbm.at[idx_v.at[pl.ds(g * CHUNK, CHUNK)]],
                    bufs[p],
                    gsems[p],
                ).wait()
                store_desc(g, p).start()
            return carry

        lax.fori_loop(0, n_blocks, block, 0)

        for g in range(n_blocks * NB, n_steps):
            p = g % NB
            pltpu.make_async_copy(
                table_hbm.at[idx_v.at[pl.ds(g * CHUNK, CHUNK)]], bufs[p], gsems[p]
            ).wait()
            store_desc(g, p).start()
            if g >= 1:
                store_desc(g - 1, (g - 1) % NB).wait()

        store_desc(n_steps - 1, (n_steps - 1) % NB).wait()

    return gather_kernel


_gather = _make_gather(VOCAB, HIDDEN, _B, _CHUNK, _NB)


@jax.jit
def kernel(input_ids, labels, embed_weight):
    del labels
    return _gather(input_ids.astype(jnp.int32), embed_weight)


# speedup vs baseline: 1.0035x; 1.0035x over previous
import functools

import jax
import jax.numpy as jnp
from jax import lax
from jax.experimental import pallas as pl
from jax.experimental.pallas import tpu as pltpu
from jax.experimental.pallas import tpu_sc as plsc

VOCAB = 100000
HIDDEN = 1024
BATCH = 4
SEQ = 4096

_B = BATCH * SEQ
_CHUNK = 16
_NB = 6


def _make_gather(V, D, B, CHUNK, NB):
    info = plsc.get_sparse_core_info()
    NC, NS = info.num_cores, info.num_subcores
    NW = NC * NS
    assert B % NW == 0
    b_per_w = B // NW
    n_steps = b_per_w // CHUNK
    n_blocks = n_steps // NB
    assert b_per_w % CHUNK == 0 and NB >= 2
    w_per_row = SEQ // b_per_w
    assert SEQ % b_per_w == 0
    mesh = plsc.VectorSubcoreMesh(core_axis_name="c", subcore_axis_name="s")

    @functools.partial(
        pl.kernel,
        mesh=mesh,
        out_type=jax.ShapeDtypeStruct((BATCH, SEQ, D), jnp.float32),
        scratch_types=[
            pltpu.VMEM((b_per_w,), jnp.int32),
            [pltpu.VMEM((CHUNK, D), jnp.float32) for _ in range(NB)],
            [pltpu.SemaphoreType.DMA for _ in range(NB)],
            [pltpu.SemaphoreType.DMA for _ in range(NB)],
        ],
    )
    def gather_kernel(ids_hbm, table_hbm, out_hbm, idx_v, bufs, gsems, ssems):
        wid = lax.axis_index("s") * NC + lax.axis_index("c")
        row = wid // w_per_row
        col = (wid % w_per_row) * b_per_w
        pltpu.sync_copy(ids_hbm.at[row, pl.ds(col, b_per_w)], idx_v)

        def gather_to(g, p):
            pltpu.async_copy(
                table_hbm.at[idx_v.at[pl.ds(g * CHUNK, CHUNK)]], bufs[p], gsems[p]
            )

        def store_desc(g, p):
            return pltpu.make_async_copy(
                bufs[p], out_hbm.at[row, pl.ds(col + g * CHUNK, CHUNK)], ssems[p]
            )

        for p in range(NB - 1):
            gather_to(p, p)

        def block(i, carry):
            for p in range(NB):
                g = i * NB + p
                q = (p - 1) % NB

                @pl.when(g >= 1)
                def _():
                    store_desc(g - 1, q).wait()

                @pl.when(g + NB - 1 < n_steps)
                def _():
                    gather_to(g + NB - 1, q)

                pltpu.make_async_copy(
                    table_hbm.at[idx_v.at[pl.ds(g * CHUNK, CHUNK)]],
                    bufs[p],
                    gsems[p],
                ).wait()
                store_desc(g, p).start()
            return carry

        lax.fori_loop(0, n_blocks, block, 0)

        for g in range(n_blocks * NB, n_steps):
            p = g % NB
            pltpu.make_async_copy(
                table_hbm.at[idx_v.at[pl.ds(g * CHUNK, CHUNK)]], bufs[p], gsems[p]
            ).wait()
            store_desc(g, p).start()
            if g >= 1:
                store_desc(g - 1, (g - 1) % NB).wait()

        store_desc(n_steps - 1, (n_steps - 1) % NB).wait()

    return gather_kernel


_gather = _make_gather(VOCAB, HIDDEN, _B, _CHUNK, _NB)


@jax.jit
def kernel(input_ids, labels, embed_weight):
    del labels
    return _gather(input_ids.astype(jnp.int32), embed_weight)


# speedup vs baseline: 1.0038x; 1.0002x over previous
import functools

import jax
import jax.numpy as jnp
from jax import lax
from jax.experimental import pallas as pl
from jax.experimental.pallas import tpu as pltpu
from jax.experimental.pallas import tpu_sc as plsc

VOCAB = 100000
HIDDEN = 1024
BATCH = 4
SEQ = 4096

_B = BATCH * SEQ
_CHUNK = 32
_NB = 3


def _make_gather(V, D, B, CHUNK, NB):
    info = plsc.get_sparse_core_info()
    NC, NS = info.num_cores, info.num_subcores
    NW = NC * NS
    assert B % NW == 0
    b_per_w = B // NW
    n_steps = b_per_w // CHUNK
    n_blocks = n_steps // NB
    assert b_per_w % CHUNK == 0 and NB >= 2
    w_per_row = SEQ // b_per_w
    assert SEQ % b_per_w == 0
    mesh = plsc.VectorSubcoreMesh(core_axis_name="c", subcore_axis_name="s")

    @functools.partial(
        pl.kernel,
        mesh=mesh,
        out_type=jax.ShapeDtypeStruct((BATCH, SEQ, D), jnp.float32),
        scratch_types=[
            pltpu.VMEM((b_per_w,), jnp.int32),
            [pltpu.VMEM((CHUNK, D), jnp.float32) for _ in range(NB)],
            [pltpu.SemaphoreType.DMA for _ in range(NB)],
            [pltpu.SemaphoreType.DMA for _ in range(NB)],
        ],
    )
    def gather_kernel(ids_hbm, table_hbm, out_hbm, idx_v, bufs, gsems, ssems):
        wid = lax.axis_index("s") * NC + lax.axis_index("c")
        row = wid // w_per_row
        col = (wid % w_per_row) * b_per_w
        pltpu.sync_copy(ids_hbm.at[row, pl.ds(col, b_per_w)], idx_v)

        def gather_to(g, p):
            pltpu.async_copy(
                table_hbm.at[idx_v.at[pl.ds(g * CHUNK, CHUNK)]], bufs[p], gsems[p]
            )

        def store_desc(g, p):
            return pltpu.make_async_copy(
                bufs[p], out_hbm.at[row, pl.ds(col + g * CHUNK, CHUNK)], ssems[p]
            )

        for p in range(NB - 1):
            gather_to(p, p)

        def block(i, carry):
            for p in range(NB):
                g = i * NB + p
                q = (p - 1) % NB

                @pl.when(g >= 1)
                def _():
                    store_desc(g - 1, q).wait()

                @pl.when(g + NB - 1 < n_steps)
                def _():
                    gather_to(g + NB - 1, q)

                pltpu.make_async_copy(
                    table_hbm.at[idx_v.at[pl.ds(g * CHUNK, CHUNK)]],
                    bufs[p],
                    gsems[p],
                ).wait()
                store_desc(g, p).start()
            return carry

        lax.fori_loop(0, n_blocks, block, 0)

        for g in range(n_blocks * NB, n_steps):
            p = g % NB
            pltpu.make_async_copy(
                table_hbm.at[idx_v.at[pl.ds(g * CHUNK, CHUNK)]], bufs[p], gsems[p]
            ).wait()
            store_desc(g, p).start()
            if g >= 1:
                store_desc(g - 1, (g - 1) % NB).wait()

        store_desc(n_steps - 1, (n_steps - 1) % NB).wait()

    return gather_kernel


_gather = _make_gather(VOCAB, HIDDEN, _B, _CHUNK, _NB)


@jax.jit
def kernel(input_ids, labels, embed_weight):
    del labels
    return _gather(input_ids.astype(jnp.int32), embed_weight)
